# Initial kernel scaffold; baseline (speedup 1.0000x reference)
import jax
import jax.numpy as jnp
from jax.experimental import pallas as pl


def kernel(points, gt_boxes, extend_gt_boxes):
    raise NotImplementedError("write your pallas kernel here")


# speedup vs baseline: 6.5612x; 6.5612x over previous
import functools

import jax
import jax.numpy as jnp
from jax.experimental import pallas as pl


def _assign_kernel(pts_ref, gtT_ref, extT_ref, cls_ref, box_ref, *, n_boxes, m_per_b):
    pts = pts_ref[...]
    gtT = gtT_ref[...]
    extT = extT_ref[...]

    bs = pts[:, 0:1].astype(jnp.int32)
    x = pts[:, 1:2]
    y = pts[:, 2:3]
    z = pts[:, 3:4]

    np_ = pts.shape[0]
    lane = jax.lax.broadcasted_iota(jnp.int32, (np_, n_boxes), 1)
    bmask = (lane // m_per_b) == bs

    def in_flags(t):
        cosa = jnp.cos(t[6:7, :])
        sina = jnp.sin(t[6:7, :])
        sx = x - t[0:1, :]
        sy = y - t[1:2, :]
        sz = z - t[2:3, :]
        lx = sx * cosa + sy * sina
        ly = -sx * sina + sy * cosa
        return ((jnp.abs(lx) <= t[3:4, :] * 0.5)
                & (jnp.abs(ly) <= t[4:5, :] * 0.5)
                & (jnp.abs(sz) <= t[5:6, :] * 0.5)
                & bmask)

    inb = in_flags(gtT)
    ine = in_flags(extT)

    fg = jnp.any(inb, axis=1, keepdims=True)
    exta = jnp.any(ine, axis=1, keepdims=True)
    ignore = jnp.logical_xor(fg, exta)
    cls_ref[...] = jnp.where(ignore, -1, jnp.where(fg, 1, 0)).astype(jnp.int32)

    hit = jnp.min(jnp.where(inb, lane, n_boxes), axis=1, keepdims=True)
    hitb = lane == hit

    enc = jnp.concatenate(
        [gtT[0:3, :],
         jnp.log(jnp.maximum(gtT[3:6, :], 1e-3)),
         jnp.cos(gtT[6:7, :]),
         jnp.sin(gtT[6:7, :])], axis=0)

    g = jnp.concatenate(
        [jnp.sum(jnp.where(hitb, enc[r:r + 1, :], 0.0), axis=1, keepdims=True)
         for r in range(8)], axis=1)
    offs = g[:, 0:3] - jnp.concatenate([x, y, z], axis=1)
    box = jnp.concatenate([offs, g[:, 3:8]], axis=1)
    box_ref[...] = box * fg.astype(jnp.float32)


def kernel(points, gt_boxes, extend_gt_boxes):
    n = points.shape[0]
    b, m, c = gt_boxes.shape
    nb = b * m
    gtT = gt_boxes.reshape(nb, c).T
    extT = extend_gt_boxes.reshape(nb, c).T
    np_ = 2000
    grid = (n // np_,)
    body = functools.partial(_assign_kernel, n_boxes=nb, m_per_b=m)
    cls, box = pl.pallas_call(
        body,
        grid=grid,
        in_specs=[
            pl.BlockSpec((np_, 4), lambda i: (i, 0)),
            pl.BlockSpec((c, nb), lambda i: (0, 0)),
            pl.BlockSpec((c, nb), lambda i: (0, 0)),
        ],
        out_specs=[
            pl.BlockSpec((np_, 1), lambda i: (i, 0)),
            pl.BlockSpec((np_, 8), lambda i: (i, 0)),
        ],
        out_shape=[
            jax.ShapeDtypeStruct((n, 1), jnp.int32),
            jax.ShapeDtypeStruct((n, 8), jnp.float32),
        ],
    )(points, gtT, extT)
    return cls[:, 0], box
